# baseline (speedup 1.0000x reference)
import jax
import jax.numpy as jnp
from jax.experimental import pallas as pl

N = 10000
DEG = 16
D = 256
BN = 400


def _body(ce_ref, al_ref, msg_ref, out_ref):
    al = al_ref[...]
    m = msg_ref[...]
    ce = ce_ref[...]
    out_ref[...] = jnp.sum(al * m, axis=1) + ce


def kernel(curr_emb, alpha, msg):
    ce_flat = curr_emb.reshape(N, DEG * D)
    grid = (N // BN,)
    return pl.pallas_call(
        _body,
        grid=grid,
        in_specs=[
            pl.BlockSpec((BN, D), lambda i: (i, 0)),
            pl.BlockSpec((BN, DEG, 1), lambda i: (i, 0, 0)),
            pl.BlockSpec((BN, DEG, D), lambda i: (i, 0, 0)),
        ],
        out_specs=pl.BlockSpec((BN, D), lambda i: (i, 0)),
        out_shape=jax.ShapeDtypeStruct((N, D), jnp.float32),
    )(ce_flat, alpha, msg)
